# baseline (speedup 1.0000x reference)
import functools

import jax
import jax.numpy as jnp
from jax import lax
from jax.experimental import pallas as pl
from jax.experimental.pallas import tpu as pltpu
from jax.experimental.pallas import tpu_sc as plsc

_NUM_CORES = 2
_NUM_SUBCORES = 16
_NUM_WORKERS = _NUM_CORES * _NUM_SUBCORES


def _sc_pair_table(tt, tail_pairs):
    D, V = tt.shape
    D2 = 2 * D
    cb = 128
    nblk = V // cb
    tail = V - nblk * cb
    bpw = (nblk + _NUM_WORKERS - 1) // _NUM_WORKERS
    nd = D // 16

    mesh = plsc.VectorSubcoreMesh(core_axis_name="c", subcore_axis_name="s")

    @functools.partial(
        pl.kernel,
        mesh=mesh,
        out_type=jax.ShapeDtypeStruct((V // 2, D2), jnp.float32),
        compiler_params=pltpu.CompilerParams(needs_layout_passes=False),
        scratch_types=[
            pltpu.VMEM((2, D, cb), jnp.float32),
            pltpu.VMEM((2, cb // 2, D2), jnp.float32),
            pltpu.SemaphoreType.DMA,
            pltpu.SemaphoreType.DMA,
            pltpu.SemaphoreType.DMA,
            pltpu.SemaphoreType.DMA,
        ],
    )
    def k(tt_hbm, tail_hbm, tp_hbm, blk, tbl, gsem0, gsem1, osem0, osem1):
        wid = lax.axis_index("s") * _NUM_CORES + lax.axis_index("c")
        gsems = (gsem0, gsem1)
        osems = (osem0, osem1)

        def block_id(i):
            return wid + i * _NUM_WORKERS

        def fire(i, buf):
            b = block_id(i)

            @pl.when(b < nblk)
            def _():
                pltpu.async_copy(
                    tt_hbm.at[:, pl.ds(pl.multiple_of(b * cb, cb), cb)],
                    blk.at[buf],
                    gsems[buf],
                )

        def transpose_and_store(i, buf):
            b = block_id(i)

            @pl.when(b < nblk)
            def _():
                pltpu.make_async_copy(
                    tt_hbm.at[:, pl.ds(0, cb)], blk.at[buf], gsems[buf]
                ).wait()

                rows = [
                    lax.iota(jnp.int32, 16) + kq * 16 for kq in range(nd)
                ]
                unroll = 4

                def pr_body(pp, carry):
                    for u in range(unroll):
                        p = pp * unroll + u
                        for half in range(2):
                            jvec = jnp.full((16,), p * 2 + half, jnp.int32)
                            for kq in range(nd):
                                vals = plsc.load_gather(
                                    blk.at[buf], [rows[kq], jvec]
                                )
                                tbl[buf, p, pl.ds(half * D + kq * 16, 16)] = vals
                    return carry

                lax.fori_loop(0, cb // 2 // unroll, pr_body, 0)
                pltpu.async_copy(
                    tbl.at[buf],
                    tp_hbm.at[
                        pl.ds(pl.multiple_of(b * (cb // 2), cb // 2), cb // 2)
                    ],
                    osems[buf],
                )

        def drain_out(i, buf):
            @pl.when((i >= 0) & (block_id(i) < nblk))
            def _():
                pltpu.make_async_copy(
                    tbl.at[buf], tp_hbm.at[pl.ds(0, cb // 2)], osems[buf]
                ).wait()

        fire(0, 0)
        bpw2 = (bpw + 1) // 2

        def body(i2, carry):
            i0 = i2 * 2
            fire(i0 + 1, 1)
            transpose_and_store(i0, 0)
            drain_out(i0 - 1, 1)
            fire(i0 + 2, 0)
            transpose_and_store(i0 + 1, 1)
            drain_out(i0, 0)
            return carry

        lax.fori_loop(0, bpw2, body, 0)
        drain_out(bpw2 * 2 - 1, 1)

        if tail:

            @pl.when(wid == 0)
            def _():
                pltpu.sync_copy(tail_hbm, tbl.at[0, pl.ds(0, tail // 2)])
                pltpu.sync_copy(
                    tbl.at[0, pl.ds(0, tail // 2)],
                    tp_hbm.at[pl.ds(nblk * (cb // 2), tail // 2)],
                )

    return k(tt, tail_pairs)


def _sc_pool(xf, lengths, tpair, B, S, D):
    D2 = 2 * D
    rpw = B // _NUM_WORKERS
    nvec = D // 16
    ch = 40
    nch_max = S // ch
    chpad = 48
    nvi = chpad // 16
    nbuf = 3

    mesh = plsc.VectorSubcoreMesh(core_axis_name="c", subcore_axis_name="s")

    @functools.partial(
        pl.kernel,
        mesh=mesh,
        out_type=jax.ShapeDtypeStruct((B, D2), jnp.float32),
        scratch_types=[
            pltpu.VMEM((rpw * S + chpad,), jnp.int32),
            pltpu.VMEM((rpw + 16,), jnp.int32),
            pltpu.VMEM((nbuf, nch_max, chpad), jnp.int32),
            pltpu.VMEM((nbuf, S, D2), jnp.float32),
            pltpu.VMEM((rpw, D2), jnp.float32),
        ]
        + [pltpu.SemaphoreType.DMA] * nbuf,
    )
    def k(xf_hbm, len_hbm, tp_hbm, out_hbm, xv, lenv, pidxv, rowsv, repv, *sems):
        wid = lax.axis_index("s") * _NUM_CORES + lax.axis_index("c")
        base = wid * rpw
        pltpu.sync_copy(xf_hbm.at[pl.ds(base * S, rpw * S)], xv.at[pl.ds(0, rpw * S)])
        pltpu.sync_copy(len_hbm.at[pl.ds(base, rpw)], lenv.at[pl.ds(0, rpw)])

        def nchunks(r):
            l = lenv[pl.ds(r, 16)][0]
            return l, (l + (ch - 1)) // ch

        def fire(r, k_buf):
            _, nch = nchunks(r)
            for c in range(nch_max):

                @pl.when(c < nch)
                def _():
                    for v in range(nvi):
                        pidxv[k_buf, c, pl.ds(v * 16, 16)] = (
                            xv[pl.ds(r * S + c * ch + v * 16, 16)] >> 1
                        )
                    pltpu.async_copy(
                        tp_hbm.at[pidxv.at[k_buf, c, pl.ds(0, ch)]],
                        rowsv.at[k_buf, pl.ds(c * ch, ch)],
                        sems[k_buf],
                    )

        def drain(r, k_buf):
            _, nch = nchunks(r)
            for c in range(nch_max):

                @pl.when(c < nch)
                def _():
                    pltpu.make_async_copy(
                        tp_hbm.at[pl.ds(0, ch)],
                        rowsv.at[k_buf, pl.ds(c * ch, ch)],
                        sems[k_buf],
                    ).wait()

        def accumulate(r, k_buf):
            l, nch = nchunks(r)

            def chunk_body(c, accs):
                j0 = c * ch
                for jj in range(ch):
                    j = j0 + jj
                    take = j < l
                    half = (xv[pl.ds(r * S + j, 16)][0] & 1) << 6
                    accs = tuple(
                        accs[q]
                        + jnp.where(
                            take,
                            rowsv[k_buf, j, pl.ds(half + q * 16, 16)],
                            0.0,
                        )
                        for q in range(nvec)
                    )
                return accs

            accs = tuple(jnp.zeros((16,), jnp.float32) for _ in range(nvec))
            accs = lax.fori_loop(0, nch, chunk_body, accs)
            inv = 1.0 / jnp.full((16,), l, jnp.float32)
            for q in range(nvec):
                repv[r, pl.ds(q * 16, 16)] = accs[q] * inv

        for k_buf in range(nbuf):
            fire(k_buf, k_buf)

        def body(i, carry):
            for k_buf in range(nbuf):
                r = i * nbuf + k_buf
                drain(r, k_buf)
                accumulate(r, k_buf)
                nxt = r + nbuf

                @pl.when(nxt < rpw)
                def _():
                    fire(nxt, k_buf)

            return carry

        lax.fori_loop(0, rpw // nbuf, body, 0)

        @pl.when(rpw % nbuf != 0)
        def _():
            for k_buf in range(rpw % nbuf):
                r = (rpw // nbuf) * nbuf + k_buf
                drain(r, k_buf)
                accumulate(r, k_buf)

        pltpu.sync_copy(repv, out_hbm.at[pl.ds(base, rpw)])

    return k(xf, lengths, tpair)


def _tc_head(rep, W, b2, D):
    B, _ = rep.shape
    C = W.shape[1]

    def body(rep_ref, w_ref, b_ref, o_ref):
        r = jnp.maximum(rep_ref[:, :D], 0.0)
        o_ref[...] = (
            lax.dot_general(
                r, w_ref[...], (((1,), (0,)), ((), ())),
                preferred_element_type=jnp.float32,
            )
            + b_ref[...]
        )

    return pl.pallas_call(
        body,
        out_shape=jax.ShapeDtypeStruct((B, C), jnp.float32),
    )(rep, W, b2)


def kernel(x, lengths, table, W, b):
    x = x.astype(jnp.int32)
    lengths = lengths.astype(jnp.int32)
    B, S = x.shape
    V, D = table.shape
    xf = x.reshape(B * S)
    nblk = V // 128
    tail = V - nblk * 128
    tail_pairs = table[nblk * 128 :].reshape(tail // 2, 2 * D)
    tpair = _sc_pair_table(table.T, tail_pairs)
    rep = _sc_pool(xf, lengths, tpair, B, S, D)
    return _tc_head(rep, W, b.reshape(1, -1), D)


# speedup vs baseline: 2.2259x; 2.2259x over previous
import functools

import jax
import jax.numpy as jnp
from jax import lax
from jax.experimental import pallas as pl
from jax.experimental.pallas import tpu as pltpu
from jax.experimental.pallas import tpu_sc as plsc

_NUM_CORES = 2
_NUM_SUBCORES = 16
_NUM_WORKERS = _NUM_CORES * _NUM_SUBCORES


def _sc_pool(x, lengths, table):
    B, S = x.shape
    _, D = table.shape
    rpw = B // _NUM_WORKERS
    nvec = D // 16
    ch = 40
    nch_max = S // ch
    nbuf = 4

    mesh = plsc.VectorSubcoreMesh(core_axis_name="c", subcore_axis_name="s")

    @functools.partial(
        pl.kernel,
        mesh=mesh,
        out_type=jax.ShapeDtypeStruct((B, D), jnp.float32),
        compiler_params=pltpu.CompilerParams(use_tc_tiling_on_sc=False),
        scratch_types=[
            pltpu.VMEM((rpw, S), jnp.int32),
            pltpu.VMEM((rpw + 16,), jnp.int32),
            pltpu.VMEM((nbuf, S, D), jnp.float32),
            pltpu.VMEM((rpw, D), jnp.float32),
        ]
        + [pltpu.SemaphoreType.DMA] * nbuf,
    )
    def k(x_hbm, len_hbm, table_hbm, out_hbm, xv, lenv, rowsv, repv, *sems):
        wid = lax.axis_index("s") * _NUM_CORES + lax.axis_index("c")
        base = wid * rpw
        pltpu.sync_copy(x_hbm.at[pl.ds(base, rpw)], xv)
        pltpu.sync_copy(len_hbm.at[pl.ds(base, rpw)], lenv.at[pl.ds(0, rpw)])

        def nchunks(r):
            l = lenv[pl.ds(r, 16)][0]
            return l, (l + (ch - 1)) // ch

        def fire(r, k_buf):
            _, nch = nchunks(r)
            for c in range(nch_max):

                @pl.when(c < nch)
                def _():
                    pltpu.async_copy(
                        table_hbm.at[xv.at[r, pl.ds(c * ch, ch)]],
                        rowsv.at[k_buf, pl.ds(c * ch, ch)],
                        sems[k_buf],
                    )

        def drain(r, k_buf):
            _, nch = nchunks(r)
            for c in range(nch_max):

                @pl.when(c < nch)
                def _():
                    pltpu.make_async_copy(
                        table_hbm.at[pl.ds(0, ch)],
                        rowsv.at[k_buf, pl.ds(c * ch, ch)],
                        sems[k_buf],
                    ).wait()

        def accumulate(r, k_buf):
            l, nch = nchunks(r)

            def chunk_body(c, accs):
                j0 = c * ch
                for jj in range(ch):
                    j = j0 + jj
                    take = j < l
                    accs = tuple(
                        accs[q]
                        + jnp.where(take, rowsv[k_buf, j, pl.ds(q * 16, 16)], 0.0)
                        for q in range(nvec)
                    )
                return accs

            accs = tuple(jnp.zeros((16,), jnp.float32) for _ in range(nvec))
            accs = lax.fori_loop(0, nch, chunk_body, accs)
            inv = 1.0 / jnp.full((16,), l, jnp.float32)
            for q in range(nvec):
                repv[r, pl.ds(q * 16, 16)] = accs[q] * inv

        for k_buf in range(nbuf):
            fire(k_buf, k_buf)

        def body(i, carry):
            for k_buf in range(nbuf):
                r = i * nbuf + k_buf
                drain(r, k_buf)
                accumulate(r, k_buf)
                nxt = r + nbuf

                @pl.when(nxt < rpw)
                def _():
                    fire(nxt, k_buf)

            return carry

        lax.fori_loop(0, rpw // nbuf, body, 0)
        pltpu.sync_copy(repv, out_hbm.at[pl.ds(base, rpw)])

    return k(x, lengths, table)


def _tc_head(rep, W, b2):
    B, _ = rep.shape
    C = W.shape[1]

    def body(rep_ref, w_ref, b_ref, o_ref):
        r = jnp.maximum(rep_ref[...], 0.0)
        o_ref[...] = (
            lax.dot_general(
                r, w_ref[...], (((1,), (0,)), ((), ())),
                preferred_element_type=jnp.float32,
            )
            + b_ref[...]
        )

    return pl.pallas_call(
        body,
        out_shape=jax.ShapeDtypeStruct((B, C), jnp.float32),
    )(rep, W, b2)


def kernel(x, lengths, table, W, b):
    x = x.astype(jnp.int32)
    lengths = lengths.astype(jnp.int32)
    rep = _sc_pool(x, lengths, table)
    return _tc_head(rep, W, b.reshape(1, -1))
